# Initial kernel scaffold; baseline (speedup 1.0000x reference)
import jax
import jax.numpy as jnp
from jax.experimental import pallas as pl


def kernel(z, zt):
    raise NotImplementedError("write your pallas kernel here")


# speedup vs baseline: 1.6288x; 1.6288x over previous
import jax
import jax.numpy as jnp
from jax import lax
from jax.experimental import pallas as pl
from jax.experimental.pallas import tpu as pltpu

_EPS = 1e-09


def _pair_counts_kernel(z_ref, zt_ref, out_ref):
    @pl.when(pl.program_id(1) == 0)
    def _zero():
        out_ref[...] = jnp.zeros_like(out_ref)

    zb = z_ref[...].astype(jnp.bfloat16)
    ztb = zt_ref[...].astype(jnp.bfloat16)
    out_ref[...] += lax.dot_general(
        zb, ztb,
        dimension_numbers=(((0,), (0,)), ((), ())),
        preferred_element_type=jnp.float32,
    )[None]


def _loss_kernel(parts_ref, loss_ref):
    P = parts_ref[0] + parts_ref[1]
    P = (P + P.T) * (0.5 / jnp.sum(P))
    P = jnp.maximum(P, _EPS)
    Pi = jnp.sum(P, axis=1, keepdims=True)
    Pj = jnp.sum(P, axis=0, keepdims=True)
    loss_ref[0, 0] = (jnp.sum(Pi * jnp.log(Pi))
                      + jnp.sum(Pj * jnp.log(Pj))
                      - jnp.sum(P * jnp.log(P)))


def kernel(z, zt):
    n, c = z.shape
    assert zt.shape == (n, c)

    tile_n = 2048
    span = 2 * tile_n
    n_pad = -(-n // span) * span
    if n_pad != n:
        pad = n_pad - n
        z = jnp.pad(z, ((0, pad), (0, 0)))
        zt = jnp.pad(zt, ((0, pad), (0, 0)))
    kt = n_pad // span

    def tile_map(h, k):
        return (h * kt + k, 0)

    partials = pl.pallas_call(
        _pair_counts_kernel,
        out_shape=jax.ShapeDtypeStruct((2, c, c), jnp.float32),
        grid=(2, kt),
        in_specs=[
            pl.BlockSpec((tile_n, c), tile_map),
            pl.BlockSpec((tile_n, c), tile_map),
        ],
        out_specs=pl.BlockSpec((1, c, c), lambda h, k: (h, 0, 0)),
        compiler_params=pltpu.CompilerParams(
            dimension_semantics=("parallel", "arbitrary"),
            vmem_limit_bytes=64 * 1024 * 1024,
        ),
        cost_estimate=pl.CostEstimate(
            flops=2 * n_pad * c * c,
            transcendentals=0,
            bytes_accessed=2 * n_pad * c * 4 + 2 * c * c * 4,
        ),
    )(z, zt)

    loss = pl.pallas_call(
        _loss_kernel,
        out_shape=jax.ShapeDtypeStruct((1, 1), jnp.float32),
        in_specs=[pl.BlockSpec((2, c, c), lambda: (0, 0, 0))],
        out_specs=pl.BlockSpec(memory_space=pltpu.MemorySpace.SMEM),
        cost_estimate=pl.CostEstimate(
            flops=8 * c * c,
            transcendentals=c * c + 2 * c,
            bytes_accessed=2 * c * c * 4 + 4,
        ),
    )(partials)
    return loss[0, 0]


# speedup vs baseline: 2.2117x; 1.3579x over previous
import jax
import jax.numpy as jnp
from jax import lax
from jax.experimental import pallas as pl
from jax.experimental.pallas import tpu as pltpu

_EPS = 1e-09


def _pair_counts_kernel(z_ref, zt_ref, out_ref):
    @pl.when(pl.program_id(1) == 0)
    def _zero():
        out_ref[...] = jnp.zeros_like(out_ref)

    zb = z_ref[...].astype(jnp.bfloat16)
    ztb = zt_ref[...].astype(jnp.bfloat16)
    out_ref[...] += lax.dot_general(
        zb, ztb,
        dimension_numbers=(((0,), (0,)), ((), ())),
        preferred_element_type=jnp.float32,
    )[None]


def _loss_kernel(parts_ref, loss_ref):
    P = parts_ref[0] + parts_ref[1]
    P = (P + P.T) * (0.5 / jnp.sum(P))
    P = jnp.maximum(P, _EPS)
    Pi = jnp.sum(P, axis=1, keepdims=True)
    Pj = jnp.sum(P, axis=0, keepdims=True)
    loss_ref[0, 0] = (jnp.sum(Pi * jnp.log(Pi))
                      + jnp.sum(Pj * jnp.log(Pj))
                      - jnp.sum(P * jnp.log(P)))


def kernel(z, zt):
    n, c = z.shape
    assert zt.shape == (n, c)

    tile_n = 4096
    span = 2 * tile_n
    n_pad = -(-n // span) * span
    if n_pad != n:
        pad = n_pad - n
        z = jnp.pad(z, ((0, pad), (0, 0)))
        zt = jnp.pad(zt, ((0, pad), (0, 0)))
    kt = n_pad // span

    def tile_map(h, k):
        return (h * kt + k, 0)

    partials = pl.pallas_call(
        _pair_counts_kernel,
        out_shape=jax.ShapeDtypeStruct((2, c, c), jnp.float32),
        grid=(2, kt),
        in_specs=[
            pl.BlockSpec((tile_n, c), tile_map),
            pl.BlockSpec((tile_n, c), tile_map),
        ],
        out_specs=pl.BlockSpec((1, c, c), lambda h, k: (h, 0, 0)),
        compiler_params=pltpu.CompilerParams(
            dimension_semantics=("parallel", "arbitrary"),
            vmem_limit_bytes=64 * 1024 * 1024,
        ),
        cost_estimate=pl.CostEstimate(
            flops=2 * n_pad * c * c,
            transcendentals=0,
            bytes_accessed=2 * n_pad * c * 4 + 2 * c * c * 4,
        ),
    )(z, zt)

    loss = pl.pallas_call(
        _loss_kernel,
        out_shape=jax.ShapeDtypeStruct((1, 1), jnp.float32),
        in_specs=[pl.BlockSpec((2, c, c), lambda: (0, 0, 0))],
        out_specs=pl.BlockSpec(memory_space=pltpu.MemorySpace.SMEM),
        cost_estimate=pl.CostEstimate(
            flops=8 * c * c,
            transcendentals=c * c + 2 * c,
            bytes_accessed=2 * c * c * 4 + 4,
        ),
    )(partials)
    return loss[0, 0]


# speedup vs baseline: 2.4095x; 1.0894x over previous
import jax
import jax.numpy as jnp
from jax import lax
from jax.experimental import pallas as pl
from jax.experimental.pallas import tpu as pltpu

_EPS = 1e-09


def _pair_counts_kernel(z_ref, zt_ref, out_ref):
    @pl.when(pl.program_id(1) == 0)
    def _zero():
        out_ref[...] = jnp.zeros_like(out_ref)

    zb = z_ref[...].astype(jnp.bfloat16)
    ztb = zt_ref[...].astype(jnp.bfloat16)
    out_ref[...] += lax.dot_general(
        zb, ztb,
        dimension_numbers=(((0,), (0,)), ((), ())),
        preferred_element_type=jnp.float32,
    )[None]


def _loss_kernel(parts_ref, loss_ref):
    P = parts_ref[0] + parts_ref[1]
    P = (P + P.T) * (0.5 / jnp.sum(P))
    P = jnp.maximum(P, _EPS)
    Pi = jnp.sum(P, axis=1, keepdims=True)
    Pj = jnp.sum(P, axis=0, keepdims=True)
    loss_ref[0, 0] = (jnp.sum(Pi * jnp.log(Pi))
                      + jnp.sum(Pj * jnp.log(Pj))
                      - jnp.sum(P * jnp.log(P)))


def kernel(z, zt):
    n, c = z.shape
    assert zt.shape == (n, c)

    tile_n = 8192
    span = 2 * tile_n
    n_pad = -(-n // span) * span
    if n_pad != n:
        pad = n_pad - n
        z = jnp.pad(z, ((0, pad), (0, 0)))
        zt = jnp.pad(zt, ((0, pad), (0, 0)))
    kt = n_pad // span

    def tile_map(h, k):
        return (h * kt + k, 0)

    partials = pl.pallas_call(
        _pair_counts_kernel,
        out_shape=jax.ShapeDtypeStruct((2, c, c), jnp.float32),
        grid=(2, kt),
        in_specs=[
            pl.BlockSpec((tile_n, c), tile_map),
            pl.BlockSpec((tile_n, c), tile_map),
        ],
        out_specs=pl.BlockSpec((1, c, c), lambda h, k: (h, 0, 0)),
        compiler_params=pltpu.CompilerParams(
            dimension_semantics=("parallel", "arbitrary"),
            vmem_limit_bytes=64 * 1024 * 1024,
        ),
        cost_estimate=pl.CostEstimate(
            flops=2 * n_pad * c * c,
            transcendentals=0,
            bytes_accessed=2 * n_pad * c * 4 + 2 * c * c * 4,
        ),
    )(z, zt)

    loss = pl.pallas_call(
        _loss_kernel,
        out_shape=jax.ShapeDtypeStruct((1, 1), jnp.float32),
        in_specs=[pl.BlockSpec((2, c, c), lambda: (0, 0, 0))],
        out_specs=pl.BlockSpec(memory_space=pltpu.MemorySpace.SMEM),
        cost_estimate=pl.CostEstimate(
            flops=8 * c * c,
            transcendentals=c * c + 2 * c,
            bytes_accessed=2 * c * c * 4 + 4,
        ),
    )(partials)
    return loss[0, 0]
